# baseline (speedup 1.0000x reference)
import functools

import jax
import jax.numpy as jnp
from jax import lax
from jax.experimental import pallas as pl
from jax.experimental.pallas import tpu as pltpu
from jax.experimental.pallas import tpu_sc as plsc

_B, _D, _F, _K = 32, 2048, 65536, 128
_TE = 1024
_NTE = _F // _TE
_TD = 2048
_NTD = _F // _TD
_NC, _NS, _L = 2, 16, 16


def _sortable_u32(v):
    b = lax.bitcast_convert_type(v, jnp.uint32)
    neg = b >= jnp.uint32(0x80000000)
    return jnp.where(neg, ~b, b | jnp.uint32(0x80000000))


def _unsort_f32(k):
    pos = k >= jnp.uint32(0x80000000)
    b = jnp.where(pos, k ^ jnp.uint32(0x80000000), ~k)
    return lax.bitcast_convert_type(b, jnp.float32)


def _enc_body(x_ref, be_ref, We_ref, key_ref):
    emb = lax.dot_general(
        x_ref[...], We_ref[...], (((1,), (1,)), ((), ())),
        preferred_element_type=jnp.float32)
    key_ref[...] = _sortable_u32(emb + be_ref[...])


def _encode(x, be2, W_enc):
    return pl.pallas_call(
        _enc_body,
        grid=(_NTE,),
        in_specs=[
            pl.BlockSpec((_B, _D), lambda i: (0, 0)),
            pl.BlockSpec((1, _TE), lambda i: (0, i)),
            pl.BlockSpec((_TE, _D), lambda i: (i, 0)),
        ],
        out_specs=pl.BlockSpec((_B, _TE), lambda i: (0, i)),
        out_shape=jax.ShapeDtypeStruct((_B, _F), jnp.uint32),
    )(x, be2, W_enc)


_mesh = plsc.VectorSubcoreMesh(core_axis_name="c", subcore_axis_name="s")


@functools.partial(
    pl.kernel,
    mesh=_mesh,
    compiler_params=pltpu.CompilerParams(needs_layout_passes=False),
    out_type=jax.ShapeDtypeStruct((_B, _L), jnp.uint32),
    scratch_types=[
        pltpu.VMEM((_F,), jnp.uint32),
        pltpu.VMEM((256 * _L,), jnp.int32),
        pltpu.VMEM((256,), jnp.int32),
        pltpu.VMEM((_L,), jnp.uint32),
    ],
)
def _sc_select(keys_hbm, thr_hbm, row_v, hist_v, tot_v, thr_v):
    w = lax.axis_index("s") * _NC + lax.axis_index("c")
    pltpu.sync_copy(keys_hbm.at[w], row_v)

    lane = lax.iota(jnp.int32, _L)
    lane_base = lane * 256
    ones = jnp.ones((_L,), jnp.int32)
    zeros16 = jnp.zeros((_L,), jnp.int32)
    true16 = jnp.ones((_L,), jnp.bool_)

    def zero_hist(b, carry):
        hist_v[pl.ds(b * _L, _L)] = zeros16
        return carry

    lax.fori_loop(0, 256, zero_hist, 0)

    def r0(i, carry):
        for u in range(8):
            v = row_v[pl.ds(i * (8 * _L) + u * _L, _L)]
            dig = lax.shift_right_logical(v, jnp.uint32(24)).astype(jnp.int32)
            plsc.addupdate_scatter(hist_v, [lane_base + dig], ones, mask=true16)
        return carry

    lax.fori_loop(0, _F // (8 * _L), r0, 0)

    def walk(k_rem):
        def red(vb, carry):
            acc = zeros16
            for l in range(_L):
                h = hist_v[pl.ds(l * 256 + vb * _L, _L)]
                hist_v[pl.ds(l * 256 + vb * _L, _L)] = zeros16
                acc = acc + h
            tot_v[pl.ds(vb * _L, _L)] = acc
            return carry

        lax.fori_loop(0, _L, red, 0)

        cum = jnp.int32(0)
        d_star = jnp.int32(0)
        found = jnp.int32(0)
        for vb in range(_L - 1, -1, -1):
            v = tot_v[pl.ds(vb * _L, _L)]
            rev = lax.rev(v, (0,))
            cs = plsc.cumsum(rev)
            ncross = jnp.sum(((cum + cs) >= k_rem).astype(jnp.int32))
            d_cand = vb * _L + ncross - 1
            take = jnp.logical_and(found == 0, ncross > 0)
            d_star = jnp.where(take, d_cand, d_star)
            found = jnp.where(take, jnp.int32(1), found)
            cum = cum + jnp.sum(v)

        above = zeros16
        for vb in range(_L):
            v = tot_v[pl.ds(vb * _L, _L)]
            binidx = vb * _L + lane
            above = above + jnp.where(binidx > d_star, v, 0)
        k_star = k_rem - jnp.sum(above)
        return d_star, k_star

    def scan_round(n_in, d_match, match_shift, hist_shift, unroll):
        dm = d_match.astype(jnp.uint32)

        def body(i, off):
            for u in range(unroll):
                base = i * (unroll * _L) + u * _L
                v = row_v[pl.ds(base, _L)]
                valid = (base + lane) < n_in
                mdig = (lax.shift_right_logical(v, jnp.uint32(match_shift))
                        & jnp.uint32(0xFF))
                m = jnp.logical_and(mdig == dm, valid)
                plsc.store_compressed(row_v.at[pl.ds(off, _L)], v, mask=m)
                dig = (lax.shift_right_logical(v, jnp.uint32(hist_shift))
                       & jnp.uint32(0xFF)).astype(jnp.int32)
                plsc.addupdate_scatter(hist_v, [lane_base + dig], ones, mask=m)
                off = off + jnp.sum(m.astype(jnp.int32))
            return off

        nv = (n_in + unroll * _L - 1) // (unroll * _L)
        return lax.fori_loop(0, nv, body, jnp.int32(0))

    d0, k1 = walk(jnp.int32(_K))
    n1 = scan_round(jnp.int32(_F), d0, 24, 16, 8)
    d1, k2 = walk(k1)
    n2 = scan_round(n1, d1, 16, 8, 1)
    d2, k3 = walk(k2)
    _ = scan_round(n2, d2, 8, 0, 1)
    d3, _ = walk(k3)

    thr = ((d0.astype(jnp.uint32) << 24) | (d1.astype(jnp.uint32) << 16)
           | (d2.astype(jnp.uint32) << 8) | d3.astype(jnp.uint32))
    thr_v[...] = jnp.full((_L,), thr, jnp.uint32)
    pltpu.sync_copy(thr_v, thr_hbm.at[w])


def _dec_body(key_ref, thr_ref, bd_ref, Wd_ref, dec_ref, enc_ref):
    j = pl.program_id(0)
    kk = key_ref[...]
    thr = thr_ref[...][:, 0:1]
    emb = _unsort_f32(kk)
    enc = jnp.where(kk >= thr, emb, jnp.float32(0.0))
    enc_ref[...] = enc
    contrib = lax.dot_general(
        enc, Wd_ref[...], (((1,), (1,)), ((), ())),
        preferred_element_type=jnp.float32)

    @pl.when(j == 0)
    def _():
        dec_ref[...] = contrib + bd_ref[...]

    @pl.when(j > 0)
    def _():
        dec_ref[...] = dec_ref[...] + contrib


def _decode(keys, thr, bd2, W_dec):
    return pl.pallas_call(
        _dec_body,
        grid=(_NTD,),
        in_specs=[
            pl.BlockSpec((_B, _TD), lambda j: (0, j)),
            pl.BlockSpec((_B, _L), lambda j: (0, 0)),
            pl.BlockSpec((1, _D), lambda j: (0, 0)),
            pl.BlockSpec((_D, _TD), lambda j: (0, j)),
        ],
        out_specs=[
            pl.BlockSpec((_B, _D), lambda j: (0, 0)),
            pl.BlockSpec((_B, _TD), lambda j: (0, j)),
        ],
        out_shape=[
            jax.ShapeDtypeStruct((_B, _D), jnp.float32),
            jax.ShapeDtypeStruct((_B, _F), jnp.float32),
        ],
    )(keys, thr, bd2, W_dec)


def kernel(x, W_enc, b_enc, W_dec, b_dec):
    be2 = b_enc.reshape(1, _F)
    bd2 = b_dec.reshape(1, _D)
    keys = _encode(x, be2, W_enc)
    thr = _sc_select(keys)
    dec, enc = _decode(keys, thr, bd2, W_dec)
    return (dec, enc, x)


# speedup vs baseline: 1.1643x; 1.1643x over previous
import functools

import jax
import jax.numpy as jnp
from jax import lax
from jax.experimental import pallas as pl
from jax.experimental.pallas import tpu as pltpu
from jax.experimental.pallas import tpu_sc as plsc

_B, _D, _F, _K = 32, 2048, 65536, 128
_TE = 1024
_NTE = _F // _TE
_TD = 2048
_NTD = _F // _TD
_NC, _NS, _L = 2, 16, 16


def _sortable_u32(v):
    b = lax.bitcast_convert_type(v, jnp.uint32)
    neg = b >= jnp.uint32(0x80000000)
    return jnp.where(neg, ~b, b | jnp.uint32(0x80000000))


def _unsort_f32(k):
    pos = k >= jnp.uint32(0x80000000)
    b = jnp.where(pos, k ^ jnp.uint32(0x80000000), ~k)
    return lax.bitcast_convert_type(b, jnp.float32)


def _enc_body(x_ref, be_ref, We_ref, key_ref):
    emb = lax.dot_general(
        x_ref[...], We_ref[...], (((1,), (1,)), ((), ())),
        preferred_element_type=jnp.float32)
    key_ref[...] = _sortable_u32(emb + be_ref[...])


def _encode(x, be2, W_enc):
    return pl.pallas_call(
        _enc_body,
        grid=(_NTE,),
        in_specs=[
            pl.BlockSpec((_B, _D), lambda i: (0, 0)),
            pl.BlockSpec((1, _TE), lambda i: (0, i)),
            pl.BlockSpec((_TE, _D), lambda i: (i, 0)),
        ],
        out_specs=pl.BlockSpec((_B, _TE), lambda i: (0, i)),
        out_shape=jax.ShapeDtypeStruct((_B, _F), jnp.uint32),
    )(x, be2, W_enc)


_mesh = plsc.VectorSubcoreMesh(core_axis_name="c", subcore_axis_name="s")


@functools.partial(
    pl.kernel,
    mesh=_mesh,
    compiler_params=pltpu.CompilerParams(needs_layout_passes=False),
    out_type=jax.ShapeDtypeStruct((_B, _L), jnp.uint32),
    scratch_types=[
        pltpu.VMEM((_F,), jnp.uint32),
        pltpu.VMEM((256 * _L,), jnp.int32),
        pltpu.VMEM((256,), jnp.int32),
        pltpu.VMEM((_L,), jnp.uint32),
    ],
)
def _sc_select(keys_hbm, thr_hbm, row_v, hist_v, tot_v, thr_v):
    w = lax.axis_index("s") * _NC + lax.axis_index("c")
    pltpu.sync_copy(keys_hbm.at[w], row_v)

    lane = lax.iota(jnp.int32, _L)
    lane_base = lane * 256
    ones = jnp.ones((_L,), jnp.int32)
    zeros16 = jnp.zeros((_L,), jnp.int32)
    true16 = jnp.ones((_L,), jnp.bool_)

    def zero_hist(b, carry):
        hist_v[pl.ds(b * _L, _L)] = zeros16
        return carry

    lax.fori_loop(0, 256, zero_hist, 0)

    def scan_hist(match_shift, pref, dig_shift):
        if pref is not None:
            pref_u = jnp.full((_L,), pref.astype(jnp.uint32), jnp.uint32)

        @plsc.parallel_loop(0, _F // _L, unroll=8)
        def body(i):
            v = row_v[pl.ds(i * _L, _L)]
            if pref is None:
                m = true16
            else:
                m = lax.shift_right_logical(v, jnp.uint32(match_shift)) == pref_u
            dig = (lax.shift_right_logical(v, jnp.uint32(dig_shift))
                   & jnp.uint32(0xFF)).astype(jnp.int32)
            plsc.addupdate_scatter(hist_v, [lane_base + dig], ones, mask=m)

    def walk(k_rem):
        def red(vb, carry):
            acc = zeros16
            for l in range(_L):
                h = hist_v[pl.ds(l * 256 + vb * _L, _L)]
                hist_v[pl.ds(l * 256 + vb * _L, _L)] = zeros16
                acc = acc + h
            tot_v[pl.ds(vb * _L, _L)] = acc
            return carry

        lax.fori_loop(0, _L, red, 0)

        cum = jnp.int32(0)
        d_star = jnp.int32(0)
        found = jnp.int32(0)
        for vb in range(_L - 1, -1, -1):
            v = tot_v[pl.ds(vb * _L, _L)]
            rev = lax.rev(v, (0,))
            cs = plsc.cumsum(rev)
            ncross = jnp.sum(((cum + cs) >= k_rem).astype(jnp.int32))
            d_cand = vb * _L + ncross - 1
            take = jnp.logical_and(found == 0, ncross > 0)
            d_star = jnp.where(take, d_cand, d_star)
            found = jnp.where(take, jnp.int32(1), found)
            cum = cum + jnp.sum(v)

        above = zeros16
        for vb in range(_L):
            v = tot_v[pl.ds(vb * _L, _L)]
            binidx = vb * _L + lane
            above = above + jnp.where(binidx > d_star, v, 0)
        k_star = k_rem - jnp.sum(above)
        return d_star, k_star

    scan_hist(0, None, 24)
    d0, k1 = walk(jnp.int32(_K))
    scan_hist(24, d0, 16)
    d1, k2 = walk(k1)
    scan_hist(16, d0 * 256 + d1, 8)
    d2, k3 = walk(k2)
    scan_hist(8, (d0 * 256 + d1) * 256 + d2, 0)
    d3, _ = walk(k3)

    thr = ((d0.astype(jnp.uint32) << 24) | (d1.astype(jnp.uint32) << 16)
           | (d2.astype(jnp.uint32) << 8) | d3.astype(jnp.uint32))
    thr_v[...] = jnp.full((_L,), thr, jnp.uint32)
    pltpu.sync_copy(thr_v, thr_hbm.at[w])


def _dec_body(key_ref, thr_ref, bd_ref, Wd_ref, dec_ref, enc_ref):
    j = pl.program_id(0)
    kk = key_ref[...]
    thr = thr_ref[...][:, 0:1]
    emb = _unsort_f32(kk)
    enc = jnp.where(kk >= thr, emb, jnp.float32(0.0))
    enc_ref[...] = enc
    contrib = lax.dot_general(
        enc, Wd_ref[...], (((1,), (1,)), ((), ())),
        preferred_element_type=jnp.float32)

    @pl.when(j == 0)
    def _():
        dec_ref[...] = contrib + bd_ref[...]

    @pl.when(j > 0)
    def _():
        dec_ref[...] = dec_ref[...] + contrib


def _decode(keys, thr, bd2, W_dec):
    return pl.pallas_call(
        _dec_body,
        grid=(_NTD,),
        in_specs=[
            pl.BlockSpec((_B, _TD), lambda j: (0, j)),
            pl.BlockSpec((_B, _L), lambda j: (0, 0)),
            pl.BlockSpec((1, _D), lambda j: (0, 0)),
            pl.BlockSpec((_D, _TD), lambda j: (0, j)),
        ],
        out_specs=[
            pl.BlockSpec((_B, _D), lambda j: (0, 0)),
            pl.BlockSpec((_B, _TD), lambda j: (0, j)),
        ],
        out_shape=[
            jax.ShapeDtypeStruct((_B, _D), jnp.float32),
            jax.ShapeDtypeStruct((_B, _F), jnp.float32),
        ],
    )(keys, thr, bd2, W_dec)


def kernel(x, W_enc, b_enc, W_dec, b_dec):
    be2 = b_enc.reshape(1, _F)
    bd2 = b_dec.reshape(1, _D)
    keys = _encode(x, be2, W_enc)
    thr = _sc_select(keys)
    dec, enc = _decode(keys, thr, bd2, W_dec)
    return (dec, enc, x)


# speedup vs baseline: 1.1739x; 1.0082x over previous
import functools

import jax
import jax.numpy as jnp
from jax import lax
from jax.experimental import pallas as pl
from jax.experimental.pallas import tpu as pltpu
from jax.experimental.pallas import tpu_sc as plsc

_B, _D, _F, _K = 32, 2048, 65536, 128
_TE = 2048
_NTE = _F // _TE
_TD = 2048
_NTD = _F // _TD
_NC, _NS, _L = 2, 16, 16


def _sortable_u32(v):
    b = lax.bitcast_convert_type(v, jnp.uint32)
    neg = b >= jnp.uint32(0x80000000)
    return jnp.where(neg, ~b, b | jnp.uint32(0x80000000))


def _unsort_f32(k):
    pos = k >= jnp.uint32(0x80000000)
    b = jnp.where(pos, k ^ jnp.uint32(0x80000000), ~k)
    return lax.bitcast_convert_type(b, jnp.float32)


def _enc_body(x_ref, be_ref, We_ref, key_ref):
    emb = lax.dot_general(
        x_ref[...], We_ref[...], (((1,), (1,)), ((), ())),
        preferred_element_type=jnp.float32)
    key_ref[...] = _sortable_u32(emb + be_ref[...])


def _encode(x, be2, W_enc):
    return pl.pallas_call(
        _enc_body,
        grid=(_NTE,),
        in_specs=[
            pl.BlockSpec((_B, _D), lambda i: (0, 0)),
            pl.BlockSpec((1, _TE), lambda i: (0, i)),
            pl.BlockSpec((_TE, _D), lambda i: (i, 0)),
        ],
        out_specs=pl.BlockSpec((_B, _TE), lambda i: (0, i)),
        out_shape=jax.ShapeDtypeStruct((_B, _F), jnp.uint32),
    )(x, be2, W_enc)


_mesh = plsc.VectorSubcoreMesh(core_axis_name="c", subcore_axis_name="s")


@functools.partial(
    pl.kernel,
    mesh=_mesh,
    compiler_params=pltpu.CompilerParams(needs_layout_passes=False),
    out_type=jax.ShapeDtypeStruct((_B, _L), jnp.uint32),
    scratch_types=[
        pltpu.VMEM((_F,), jnp.uint32),
        pltpu.VMEM((256 * _L,), jnp.int32),
        pltpu.VMEM((256,), jnp.int32),
        pltpu.VMEM((_L,), jnp.uint32),
        pltpu.SemaphoreType.DMA,
    ],
)
def _sc_select(keys_hbm, thr_hbm, row_v, hist_v, tot_v, thr_v, dma_sem):
    w = lax.axis_index("s") * _NC + lax.axis_index("c")
    nq = 4
    q = _F // nq
    copies = [
        pltpu.async_copy(keys_hbm.at[w, pl.ds(ci * q, q)],
                         row_v.at[pl.ds(ci * q, q)], dma_sem)
        for ci in range(nq)
    ]

    lane = lax.iota(jnp.int32, _L)
    lane_base = lane * 256
    ones = jnp.ones((_L,), jnp.int32)
    zeros16 = jnp.zeros((_L,), jnp.int32)
    true16 = jnp.ones((_L,), jnp.bool_)

    def zero_hist(b, carry):
        hist_v[pl.ds(b * _L, _L)] = zeros16
        return carry

    lax.fori_loop(0, 256, zero_hist, 0)

    def scan_hist(match_shift, pref, dig_shift, lo=0, hi=_F // _L):
        if pref is not None:
            pref_u = jnp.full((_L,), pref.astype(jnp.uint32), jnp.uint32)

        @plsc.parallel_loop(lo, hi, unroll=8)
        def body(i):
            v = row_v[pl.ds(i * _L, _L)]
            if pref is None:
                m = true16
            else:
                m = lax.shift_right_logical(v, jnp.uint32(match_shift)) == pref_u
            dig = (lax.shift_right_logical(v, jnp.uint32(dig_shift))
                   & jnp.uint32(0xFF)).astype(jnp.int32)
            plsc.addupdate_scatter(hist_v, [lane_base + dig], ones, mask=m)

    def walk(k_rem):
        def red(vb, carry):
            acc = zeros16
            for l in range(_L):
                h = hist_v[pl.ds(l * 256 + vb * _L, _L)]
                hist_v[pl.ds(l * 256 + vb * _L, _L)] = zeros16
                acc = acc + h
            tot_v[pl.ds(vb * _L, _L)] = acc
            return carry

        lax.fori_loop(0, _L, red, 0)

        cum = jnp.int32(0)
        d_star = jnp.int32(0)
        found = jnp.int32(0)
        for vb in range(_L - 1, -1, -1):
            v = tot_v[pl.ds(vb * _L, _L)]
            rev = lax.rev(v, (0,))
            cs = plsc.cumsum(rev)
            ncross = jnp.sum(((cum + cs) >= k_rem).astype(jnp.int32))
            d_cand = vb * _L + ncross - 1
            take = jnp.logical_and(found == 0, ncross > 0)
            d_star = jnp.where(take, d_cand, d_star)
            found = jnp.where(take, jnp.int32(1), found)
            cum = cum + jnp.sum(v)

        above = zeros16
        for vb in range(_L):
            v = tot_v[pl.ds(vb * _L, _L)]
            binidx = vb * _L + lane
            above = above + jnp.where(binidx > d_star, v, 0)
        k_star = k_rem - jnp.sum(above)
        return d_star, k_star

    vq = _F // _L // nq
    for ci in range(nq):
        copies[ci].wait()
        scan_hist(0, None, 24, lo=ci * vq, hi=(ci + 1) * vq)
    d0, k1 = walk(jnp.int32(_K))
    scan_hist(24, d0, 16)
    d1, k2 = walk(k1)
    scan_hist(16, d0 * 256 + d1, 8)
    d2, k3 = walk(k2)
    scan_hist(8, (d0 * 256 + d1) * 256 + d2, 0)
    d3, _ = walk(k3)

    thr = ((d0.astype(jnp.uint32) << 24) | (d1.astype(jnp.uint32) << 16)
           | (d2.astype(jnp.uint32) << 8) | d3.astype(jnp.uint32))
    thr_v[...] = jnp.full((_L,), thr, jnp.uint32)
    pltpu.sync_copy(thr_v, thr_hbm.at[w])


def _dec_body(key_ref, thr_ref, bd_ref, Wd_ref, dec_ref, enc_ref):
    j = pl.program_id(0)
    kk = key_ref[...]
    thr = thr_ref[...][:, 0:1]
    emb = _unsort_f32(kk)
    enc = jnp.where(kk >= thr, emb, jnp.float32(0.0))
    enc_ref[...] = enc
    contrib = lax.dot_general(
        enc, Wd_ref[...], (((1,), (1,)), ((), ())),
        preferred_element_type=jnp.float32)

    @pl.when(j == 0)
    def _():
        dec_ref[...] = contrib + bd_ref[...]

    @pl.when(j > 0)
    def _():
        dec_ref[...] = dec_ref[...] + contrib


def _decode(keys, thr, bd2, W_dec):
    return pl.pallas_call(
        _dec_body,
        grid=(_NTD,),
        in_specs=[
            pl.BlockSpec((_B, _TD), lambda j: (0, j)),
            pl.BlockSpec((_B, _L), lambda j: (0, 0)),
            pl.BlockSpec((1, _D), lambda j: (0, 0)),
            pl.BlockSpec((_D, _TD), lambda j: (0, j)),
        ],
        out_specs=[
            pl.BlockSpec((_B, _D), lambda j: (0, 0)),
            pl.BlockSpec((_B, _TD), lambda j: (0, j)),
        ],
        out_shape=[
            jax.ShapeDtypeStruct((_B, _D), jnp.float32),
            jax.ShapeDtypeStruct((_B, _F), jnp.float32),
        ],
    )(keys, thr, bd2, W_dec)


def kernel(x, W_enc, b_enc, W_dec, b_dec):
    be2 = b_enc.reshape(1, _F)
    bd2 = b_dec.reshape(1, _D)
    keys = _encode(x, be2, W_enc)
    thr = _sc_select(keys)
    dec, enc = _decode(keys, thr, bd2, W_dec)
    return (dec, enc, x)


# speedup vs baseline: 1.1765x; 1.0023x over previous
import functools

import jax
import jax.numpy as jnp
from jax import lax
from jax.experimental import pallas as pl
from jax.experimental.pallas import tpu as pltpu
from jax.experimental.pallas import tpu_sc as plsc

_B, _D, _F, _K = 32, 2048, 65536, 128
_TE = 2048
_NTE = _F // _TE
_TD = 2048
_NTD = _F // _TD
_NC, _NS, _L = 2, 16, 16


def _sortable_u32(v):
    b = lax.bitcast_convert_type(v, jnp.uint32)
    neg = b >= jnp.uint32(0x80000000)
    return jnp.where(neg, ~b, b | jnp.uint32(0x80000000))


def _unsort_f32(k):
    pos = k >= jnp.uint32(0x80000000)
    b = jnp.where(pos, k ^ jnp.uint32(0x80000000), ~k)
    return lax.bitcast_convert_type(b, jnp.float32)


def _enc_body(x_ref, be_ref, We_ref, key_ref):
    emb = lax.dot_general(
        x_ref[...], We_ref[...], (((1,), (1,)), ((), ())),
        preferred_element_type=jnp.float32)
    key_ref[...] = _sortable_u32(emb + be_ref[...])


def _encode(x, be2, W_enc):
    return pl.pallas_call(
        _enc_body,
        grid=(_NTE,),
        in_specs=[
            pl.BlockSpec((_B, _D), lambda i: (0, 0)),
            pl.BlockSpec((1, _TE), lambda i: (0, i)),
            pl.BlockSpec((_TE, _D), lambda i: (i, 0)),
        ],
        out_specs=pl.BlockSpec((_B, _TE), lambda i: (0, i)),
        out_shape=jax.ShapeDtypeStruct((_B, _F), jnp.uint32),
    )(x, be2, W_enc)


_mesh = plsc.VectorSubcoreMesh(core_axis_name="c", subcore_axis_name="s")


@functools.partial(
    pl.kernel,
    mesh=_mesh,
    compiler_params=pltpu.CompilerParams(needs_layout_passes=False),
    out_type=jax.ShapeDtypeStruct((_B, _L), jnp.uint32),
    scratch_types=[
        pltpu.VMEM((_F,), jnp.uint32),
        pltpu.VMEM((256 * _L,), jnp.int32),
        pltpu.VMEM((256,), jnp.int32),
        pltpu.VMEM((_L,), jnp.uint32),
        pltpu.SemaphoreType.DMA,
    ],
)
def _sc_select(keys_hbm, thr_hbm, row_v, hist_v, tot_v, thr_v, dma_sem):
    w = lax.axis_index("s") * _NC + lax.axis_index("c")
    nq = 4
    q = _F // nq
    copies = [
        pltpu.async_copy(keys_hbm.at[w, pl.ds(ci * q, q)],
                         row_v.at[pl.ds(ci * q, q)], dma_sem)
        for ci in range(nq)
    ]

    lane = lax.iota(jnp.int32, _L)
    lane_base = lane * 256
    ones = jnp.ones((_L,), jnp.int32)
    zeros16 = jnp.zeros((_L,), jnp.int32)
    true16 = jnp.ones((_L,), jnp.bool_)

    def zero_hist(b, carry):
        hist_v[pl.ds(b * _L, _L)] = zeros16
        return carry

    lax.fori_loop(0, 256, zero_hist, 0)

    def scan_hist(match_shift, pref, dig_shift, lo=0, hi=_F // _L):
        if pref is not None:
            pref_u = jnp.full((_L,), pref.astype(jnp.uint32), jnp.uint32)

        @plsc.parallel_loop(lo, hi, unroll=8)
        def body(i):
            v = row_v[pl.ds(i * _L, _L)]
            if pref is None:
                m = true16
            else:
                m = lax.shift_right_logical(v, jnp.uint32(match_shift)) == pref_u
            dig = (lax.shift_right_logical(v, jnp.uint32(dig_shift))
                   & jnp.uint32(0xFF)).astype(jnp.int32)
            plsc.addupdate_scatter(hist_v, [lane_base + dig], ones, mask=m)

    def walk(k_rem):
        def red(vb, carry):
            acc = zeros16
            for l in range(_L):
                h = hist_v[pl.ds(l * 256 + vb * _L, _L)]
                hist_v[pl.ds(l * 256 + vb * _L, _L)] = zeros16
                acc = acc + h
            tot_v[pl.ds(vb * _L, _L)] = acc
            return carry

        lax.fori_loop(0, _L, red, 0)

        def suf(t, carry):
            cum, d_star, found = carry
            vb = _L - 1 - t
            v = tot_v[pl.ds(vb * _L, _L)]
            rev = lax.rev(v, (0,))
            cs = plsc.cumsum(rev)
            ncross = jnp.sum(((cum + cs) >= k_rem).astype(jnp.int32))
            d_cand = vb * _L + ncross - 1
            take = jnp.logical_and(found == 0, ncross > 0)
            d_star = jnp.where(take, d_cand, d_star)
            found = jnp.where(take, jnp.int32(1), found)
            return cum + jnp.sum(v), d_star, found

        _, d_star, _ = lax.fori_loop(
            0, _L, suf, (jnp.int32(0), jnp.int32(0), jnp.int32(0)))

        def abv(vb, above):
            v = tot_v[pl.ds(vb * _L, _L)]
            binidx = vb * _L + lane
            return above + jnp.where(binidx > d_star, v, 0)

        above = lax.fori_loop(0, _L, abv, zeros16)
        k_star = k_rem - jnp.sum(above)
        return d_star, k_star

    vq = _F // _L // nq
    for ci in range(nq):
        copies[ci].wait()
        scan_hist(0, None, 24, lo=ci * vq, hi=(ci + 1) * vq)
    d0, k1 = walk(jnp.int32(_K))
    scan_hist(24, d0, 16)
    d1, k2 = walk(k1)
    scan_hist(16, d0 * 256 + d1, 8)
    d2, k3 = walk(k2)
    scan_hist(8, (d0 * 256 + d1) * 256 + d2, 0)
    d3, _ = walk(k3)

    thr = ((d0.astype(jnp.uint32) << 24) | (d1.astype(jnp.uint32) << 16)
           | (d2.astype(jnp.uint32) << 8) | d3.astype(jnp.uint32))
    thr_v[...] = jnp.full((_L,), thr, jnp.uint32)
    pltpu.sync_copy(thr_v, thr_hbm.at[w])


def _dec_body(key_ref, thr_ref, bd_ref, Wd_ref, dec_ref, enc_ref):
    j = pl.program_id(0)
    kk = key_ref[...]
    thr = thr_ref[...][:, 0:1]
    emb = _unsort_f32(kk)
    enc = jnp.where(kk >= thr, emb, jnp.float32(0.0))
    enc_ref[...] = enc
    contrib = lax.dot_general(
        enc, Wd_ref[...], (((1,), (1,)), ((), ())),
        preferred_element_type=jnp.float32)

    @pl.when(j == 0)
    def _():
        dec_ref[...] = contrib + bd_ref[...]

    @pl.when(j > 0)
    def _():
        dec_ref[...] = dec_ref[...] + contrib


def _decode(keys, thr, bd2, W_dec):
    return pl.pallas_call(
        _dec_body,
        grid=(_NTD,),
        in_specs=[
            pl.BlockSpec((_B, _TD), lambda j: (0, j)),
            pl.BlockSpec((_B, _L), lambda j: (0, 0)),
            pl.BlockSpec((1, _D), lambda j: (0, 0)),
            pl.BlockSpec((_D, _TD), lambda j: (0, j)),
        ],
        out_specs=[
            pl.BlockSpec((_B, _D), lambda j: (0, 0)),
            pl.BlockSpec((_B, _TD), lambda j: (0, j)),
        ],
        out_shape=[
            jax.ShapeDtypeStruct((_B, _D), jnp.float32),
            jax.ShapeDtypeStruct((_B, _F), jnp.float32),
        ],
    )(keys, thr, bd2, W_dec)


def kernel(x, W_enc, b_enc, W_dec, b_dec):
    be2 = b_enc.reshape(1, _F)
    bd2 = b_dec.reshape(1, _D)
    keys = _encode(x, be2, W_enc)
    thr = _sc_select(keys)
    dec, enc = _decode(keys, thr, bd2, W_dec)
    return (dec, enc, x)
